# baseline (speedup 1.0000x reference)
import functools

import jax
import jax.numpy as jnp
from jax import lax
from jax.experimental import pallas as pl
from jax.experimental.pallas import tpu as pltpu
from jax.experimental.pallas import tpu_sc as plsc

N = 100000
D = 128
G = 512
BA = 4000
NB = N // BA

NC = 2
NS = 16
NW = NC * NS
CH = 3136
BLK = 112
NFULL = CH // BLK
LASTW_FULL = (N - 31 * CH) // BLK
TAIL = N - 31 * CH - LASTW_FULL * BLK
NBUF = 2


def _pass_a(z_ref, w1_ref, b1_ref, w2_ref, b2_ref, a_ref, m_ref):
    i = pl.program_id(0)
    h = jnp.tanh(
        lax.dot_general(z_ref[...], w1_ref[...], (((1,), (1,)), ((), ())),
                        preferred_element_type=jnp.float32)
        + b1_ref[...][None, :])
    al = lax.dot_general(h, w2_ref[...], (((1,), (1,)), ((), ())),
                         preferred_element_type=jnp.float32)
    a_ref[...] = al + b2_ref[0, 0]

    @pl.when(i == 0)
    def _():
        m_ref[0, 0] = -jnp.inf

    m_ref[0, 0] = jnp.maximum(m_ref[0, 0], jnp.max(al))


def _sc_body(z_hbm, batch_hbm, a_hbm, m_hbm, outz_hbm, outs_hbm, *rest):
    zbs = rest[0:NBUF]
    idxbs = rest[NBUF:2 * NBUF]
    abs_ = rest[2 * NBUF:3 * NBUF]
    mv, acct, accse = rest[3 * NBUF:3 * NBUF + 3]
    sins = rest[3 * NBUF + 3:4 * NBUF + 3]
    c = lax.axis_index("c")
    s = lax.axis_index("s")
    wid = s * NC + c

    def start_in(base, k, rows):
        pltpu.async_copy(z_hbm.at[pl.ds(base, rows)],
                         zbs[k].at[pl.ds(0, rows)], sins[k])
        pltpu.async_copy(a_hbm.at[pl.ds(base, rows)],
                         abs_[k].at[pl.ds(0, rows)], sins[k])
        pltpu.async_copy(batch_hbm.at[pl.ds(base, rows)],
                         idxbs[k].at[pl.ds(0, rows)], sins[k])

    def wait_in(base, k, rows):
        pltpu.make_async_copy(z_hbm.at[pl.ds(base, rows)],
                              zbs[k].at[pl.ds(0, rows)], sins[k]).wait()
        pltpu.make_async_copy(a_hbm.at[pl.ds(base, rows)],
                              abs_[k].at[pl.ds(0, rows)], sins[k]).wait()
        pltpu.make_async_copy(batch_hbm.at[pl.ds(base, rows)],
                              idxbs[k].at[pl.ds(0, rows)], sins[k]).wait()

    def zrow(g, _):
        for f in range(8):
            acct[g, pl.ds(f * 16, 16)] = jnp.zeros((16,), jnp.float32)
        return 0

    def zrow_se(g, _):
        for f in range(8):
            accse[g, pl.ds(f * 16, 16)] = jnp.zeros((16,), jnp.float32)
        return 0

    lax.fori_loop(0, G, zrow, 0)
    lax.fori_loop(0, G // 8, zrow_se, 0)
    pltpu.sync_copy(m_hbm, mv)

    nblk = jnp.where(wid < NW - 1, NFULL, LASTW_FULL)
    base_w = wid * CH

    for k in range(NBUF):
        start_in(base_w + k * BLK, k, BLK)
    m_vec = mv[...]

    def accum(k, rows):
        zb, idxb, ab = zbs[k], idxbs[k], abs_[k]

        def group(g, _):
            ev = jnp.exp(ab[pl.ds(g * 16, 16)] - m_vec)
            gv = idxb[pl.ds(g * 16, 16)]
            for j in range(16):
                r = g * 16 + j
                w = jnp.broadcast_to(lax.slice(ev, (j,), (j + 1,)), (16,))
                gi = gv[j]
                se_off = (gi % 8) * 16
                accse[gi // 8, pl.ds(se_off, 16)] = (
                    accse[gi // 8, pl.ds(se_off, 16)] + w)
                for f in range(8):
                    acct[gi, pl.ds(f * 16, 16)] = (
                        acct[gi, pl.ds(f * 16, 16)]
                        + zb[r, pl.ds(f * 16, 16)] * w)
            return 0

        lax.fori_loop(0, rows // 16, group, 0)

    def superstep(t, _):
        for k in range(NBUF):
            base = base_w + (NBUF * t + k) * BLK
            wait_in(base, k, BLK)
            accum(k, BLK)

            @pl.when(t < nblk // NBUF - 1)
            def _():
                start_in(base + NBUF * BLK, k, BLK)

        return 0

    lax.fori_loop(0, nblk // NBUF, superstep, 0)

    @pl.when(wid == NW - 1)
    def _():
        base = (NW - 1) * CH + LASTW_FULL * BLK
        start_in(base, 0, TAIL)
        wait_in(base, 0, TAIL)
        accum(0, TAIL)

    pltpu.sync_copy(acct, outz_hbm.at[wid])
    pltpu.sync_copy(accse, outs_hbm.at[wid])


def _pass_c(outz_ref, outs_ref, out_ref):
    sz = jnp.sum(outz_ref[...], axis=0)
    s1 = jnp.sum(outs_ref[:, :, 0:1], axis=0) + 1e-8
    out_ref[...] = sz / s1


def kernel(z, batch, W1, b1, W2, b2):
    batch = batch.astype(jnp.int32)
    b2_2d = b2.reshape(1, 1)

    a2d, m = pl.pallas_call(
        _pass_a,
        grid=(NB,),
        in_specs=[
            pl.BlockSpec((BA, D), lambda i: (i, 0)),
            pl.BlockSpec((D, D), lambda i: (0, 0)),
            pl.BlockSpec((D,), lambda i: (0,)),
            pl.BlockSpec((1, D), lambda i: (0, 0)),
            pl.BlockSpec((1, 1), lambda i: (0, 0), memory_space=pltpu.SMEM),
        ],
        out_specs=[
            pl.BlockSpec((BA, 1), lambda i: (i, 0)),
            pl.BlockSpec((1, 1), lambda i: (0, 0), memory_space=pltpu.SMEM),
        ],
        out_shape=[
            jax.ShapeDtypeStruct((N, 1), jnp.float32),
            jax.ShapeDtypeStruct((1, 1), jnp.float32),
        ],
    )(z, W1, b1, W2, b2_2d)

    a = a2d.reshape(N)
    m16 = jnp.broadcast_to(m.reshape(1), (16,))

    sc = functools.partial(
        pl.kernel,
        out_type=[
            jax.ShapeDtypeStruct((NW, G, D), jnp.float32),
            jax.ShapeDtypeStruct((NW, G // 8, D), jnp.float32),
        ],
        mesh=plsc.VectorSubcoreMesh(core_axis_name="c", subcore_axis_name="s"),
        scratch_types=(
            [pltpu.VMEM((BLK, D), jnp.float32)] * NBUF
            + [pltpu.VMEM((BLK,), jnp.int32)] * NBUF
            + [pltpu.VMEM((BLK,), jnp.float32)] * NBUF
            + [
                pltpu.VMEM((16,), jnp.float32),
                pltpu.VMEM((G, D), jnp.float32),
                pltpu.VMEM((G // 8, D), jnp.float32),
            ]
            + [pltpu.SemaphoreType.DMA] * NBUF
        ),
    )(_sc_body)
    outz, outs = sc(z, batch, a, m16)
    outs = outs.reshape(NW, G, NS)

    return pl.pallas_call(
        _pass_c,
        in_specs=[
            pl.BlockSpec((NW, G, D), lambda: (0, 0, 0)),
            pl.BlockSpec((NW, G, NS), lambda: (0, 0, 0)),
        ],
        out_specs=pl.BlockSpec((G, D), lambda: (0, 0)),
        out_shape=jax.ShapeDtypeStruct((G, D), jnp.float32),
    )(outz, outs)


# speedup vs baseline: 1.7096x; 1.7096x over previous
import functools

import jax
import jax.numpy as jnp
from jax import lax
from jax.experimental import pallas as pl
from jax.experimental.pallas import tpu as pltpu
from jax.experimental.pallas import tpu_sc as plsc

N = 100000
D = 128
G = 512
BA = 4000
NB = N // BA

NC = 2
NS = 16
NW = NC * NS
CH = 3136
BLK = 112
NFULL = CH // BLK
LASTW_FULL = (N - 31 * CH) // BLK
TAIL = N - 31 * CH - LASTW_FULL * BLK
NBUF = 2


def _pass_a(z_ref, w1_ref, b1_ref, w2_ref, b2_ref, a_ref, m_ref):
    i = pl.program_id(0)
    h = jnp.tanh(
        lax.dot_general(z_ref[...], w1_ref[...], (((1,), (1,)), ((), ())),
                        preferred_element_type=jnp.float32)
        + b1_ref[...][None, :])
    al = lax.dot_general(h, w2_ref[...], (((1,), (1,)), ((), ())),
                         preferred_element_type=jnp.float32)
    a_ref[...] = al + b2_ref[0, 0]

    @pl.when(i == 0)
    def _():
        m_ref[0, 0] = -jnp.inf

    m_ref[0, 0] = jnp.maximum(m_ref[0, 0], jnp.max(al))


def _sc_body(z_hbm, batch_hbm, a_hbm, m_hbm, outz_hbm, outs_hbm, *rest):
    zbs = rest[0:NBUF]
    idxbs = rest[NBUF:2 * NBUF]
    abs_ = rest[2 * NBUF:3 * NBUF]
    mv, acct, accse = rest[3 * NBUF:3 * NBUF + 3]
    sins = rest[3 * NBUF + 3:4 * NBUF + 3]
    c = lax.axis_index("c")
    s = lax.axis_index("s")
    wid = s * NC + c

    def start_in(base, k, rows):
        pltpu.async_copy(z_hbm.at[pl.ds(base, rows)],
                         zbs[k].at[pl.ds(0, rows)], sins[k])
        pltpu.async_copy(a_hbm.at[pl.ds(base, rows)],
                         abs_[k].at[pl.ds(0, rows)], sins[k])
        pltpu.async_copy(batch_hbm.at[pl.ds(base, rows)],
                         idxbs[k].at[pl.ds(0, rows)], sins[k])

    def wait_in(base, k, rows):
        pltpu.make_async_copy(z_hbm.at[pl.ds(base, rows)],
                              zbs[k].at[pl.ds(0, rows)], sins[k]).wait()
        pltpu.make_async_copy(a_hbm.at[pl.ds(base, rows)],
                              abs_[k].at[pl.ds(0, rows)], sins[k]).wait()
        pltpu.make_async_copy(batch_hbm.at[pl.ds(base, rows)],
                              idxbs[k].at[pl.ds(0, rows)], sins[k]).wait()

    def zrow(g, _):
        for f in range(8):
            acct[g, pl.ds(f * 16, 16)] = jnp.zeros((16,), jnp.float32)
        return 0

    def zrow_se(g, _):
        for f in range(8):
            accse[g, pl.ds(f * 16, 16)] = jnp.zeros((16,), jnp.float32)
        return 0

    lax.fori_loop(0, G, zrow, 0)
    lax.fori_loop(0, G // 8, zrow_se, 0)
    pltpu.sync_copy(m_hbm, mv)

    nblk = jnp.where(wid < NW - 1, NFULL, LASTW_FULL)
    base_w = wid * CH

    for k in range(NBUF):
        start_in(base_w + k * BLK, k, BLK)
    m_vec = mv[...]

    def flush(g_cur, acc_se, acc):
        se_off = (g_cur % 8) * 16
        accse[g_cur // 8, pl.ds(se_off, 16)] = (
            accse[g_cur // 8, pl.ds(se_off, 16)] + acc_se)
        for f in range(8):
            acct[g_cur, pl.ds(f * 16, 16)] = (
                acct[g_cur, pl.ds(f * 16, 16)] + acc[f])

    def accum(k, rows, carry, wscale):
        zb, idxb, ab = zbs[k], idxbs[k], abs_[k]

        def group(g, carry):
            g_cur, acc_se, acc = carry
            ev = jnp.exp(ab[pl.ds(g * 16, 16)] - m_vec) * wscale
            gv = idxb[pl.ds(g * 16, 16)]
            for j in range(16):
                r = g * 16 + j
                w = jnp.broadcast_to(lax.slice(ev, (j,), (j + 1,)), (16,))
                gi = gv[j]
                boundary = gi != g_cur

                @pl.when(boundary)
                def _(g_cur=g_cur, acc_se=acc_se, acc=acc):
                    flush(g_cur, acc_se, acc)

                cf = [zb[r, pl.ds(f * 16, 16)] * w for f in range(8)]
                acc = [jnp.where(boundary, cf[f], acc[f] + cf[f])
                       for f in range(8)]
                acc_se = jnp.where(boundary, w, acc_se + w)
                g_cur = gi
            return g_cur, acc_se, acc

        return lax.fori_loop(0, rows // 16, group, carry)

    def superstep(t, carry):
        for k in range(NBUF):
            base = base_w + (NBUF * t + k) * BLK
            wait_in(base, k, BLK)
            carry = accum(k, BLK, carry, 1.0)

            @pl.when(t < nblk // NBUF - 1)
            def _():
                start_in(base + NBUF * BLK, k, BLK)

        return carry

    zero16 = jnp.zeros((16,), jnp.float32)
    carry0 = (jnp.int32(0), zero16, [zero16] * 8)
    carry = lax.fori_loop(0, nblk // NBUF, superstep, carry0)

    wtail = jnp.where(wid == NW - 1, 1.0, 0.0)
    tail_base = (NW - 1) * CH + LASTW_FULL * BLK
    start_in(tail_base, 0, TAIL)
    wait_in(tail_base, 0, TAIL)
    g_cur, acc_se, acc = accum(0, TAIL, carry, wtail)
    flush(g_cur, acc_se, acc)

    pltpu.sync_copy(acct, outz_hbm.at[wid])
    pltpu.sync_copy(accse, outs_hbm.at[wid])


def _pass_c(outz_ref, outs_ref, out_ref):
    sz = jnp.sum(outz_ref[...], axis=0)
    s1 = jnp.sum(outs_ref[:, :, 0:1], axis=0) + 1e-8
    out_ref[...] = sz / s1


def kernel(z, batch, W1, b1, W2, b2):
    batch = batch.astype(jnp.int32)
    b2_2d = b2.reshape(1, 1)

    a2d, m = pl.pallas_call(
        _pass_a,
        grid=(NB,),
        in_specs=[
            pl.BlockSpec((BA, D), lambda i: (i, 0)),
            pl.BlockSpec((D, D), lambda i: (0, 0)),
            pl.BlockSpec((D,), lambda i: (0,)),
            pl.BlockSpec((1, D), lambda i: (0, 0)),
            pl.BlockSpec((1, 1), lambda i: (0, 0), memory_space=pltpu.SMEM),
        ],
        out_specs=[
            pl.BlockSpec((BA, 1), lambda i: (i, 0)),
            pl.BlockSpec((1, 1), lambda i: (0, 0), memory_space=pltpu.SMEM),
        ],
        out_shape=[
            jax.ShapeDtypeStruct((N, 1), jnp.float32),
            jax.ShapeDtypeStruct((1, 1), jnp.float32),
        ],
    )(z, W1, b1, W2, b2_2d)

    a = a2d.reshape(N)
    m16 = jnp.broadcast_to(m.reshape(1), (16,))

    sc = functools.partial(
        pl.kernel,
        out_type=[
            jax.ShapeDtypeStruct((NW, G, D), jnp.float32),
            jax.ShapeDtypeStruct((NW, G // 8, D), jnp.float32),
        ],
        mesh=plsc.VectorSubcoreMesh(core_axis_name="c", subcore_axis_name="s"),
        scratch_types=(
            [pltpu.VMEM((BLK, D), jnp.float32)] * NBUF
            + [pltpu.VMEM((BLK,), jnp.int32)] * NBUF
            + [pltpu.VMEM((BLK,), jnp.float32)] * NBUF
            + [
                pltpu.VMEM((16,), jnp.float32),
                pltpu.VMEM((G, D), jnp.float32),
                pltpu.VMEM((G // 8, D), jnp.float32),
            ]
            + [pltpu.SemaphoreType.DMA] * NBUF
        ),
    )(_sc_body)
    outz, outs = sc(z, batch, a, m16)
    outs = outs.reshape(NW, G, NS)

    return pl.pallas_call(
        _pass_c,
        in_specs=[
            pl.BlockSpec((NW, G, D), lambda: (0, 0, 0)),
            pl.BlockSpec((NW, G, NS), lambda: (0, 0, 0)),
        ],
        out_specs=pl.BlockSpec((G, D), lambda: (0, 0)),
        out_shape=jax.ShapeDtypeStruct((G, D), jnp.float32),
    )(outz, outs)
